# baseline (speedup 1.0000x reference)
import numpy as np
import jax
import jax.numpy as jnp
from jax.experimental import pallas as pl
from jax.experimental.pallas import tpu as pltpu

VOCAB = 4096
CVAE = 32
V_PATCH = (1, 2, 4, 8, 16)
BETA = 0.25
QRESI = 0.5
NPHI = 4
B, H, W = 128, 16, 16
SN = len(V_PATCH)
S = H * W
IMG = 4
NBLK = B // IMG
LANES = IMG * CVAE
NROW = S * IMG

_ticks = np.linspace(1.0 / 3.0 / NPHI, 1.0 - 1.0 / 3.0 / NPHI, NPHI)
PIDX = tuple(int(np.argmin(np.abs(_ticks - si / (SN - 1)))) for si in range(SN))


def _pool_matrix(pn):
    fh = H // pn
    P = np.zeros((pn * pn, S), dtype=np.float32)
    for y in range(H):
        for x in range(W):
            P[(y // fh) * pn + (x // fh), y * W + x] = 1.0 / (fh * fh)
    return P


def _resize_matrix(pn):
    A = jax.image.resize(jnp.eye(pn, dtype=jnp.float32), (H, pn), method="cubic")
    return np.asarray(A)


def _split3_np(a):
    a = np.asarray(a, np.float32)
    a1 = np.asarray(a.astype(jnp.bfloat16), np.float32)
    r = (a - a1).astype(np.float32)
    a2 = np.asarray(r.astype(jnp.bfloat16), np.float32)
    a3 = np.asarray((r - a2).astype(np.float32).astype(jnp.bfloat16), np.float32)
    return a1, a2, a3


_POOL_PNS = (1, 2, 4, 8)
_UP_PNS = (2, 4, 8)
_P_NP = {pn: _pool_matrix(pn) for pn in _POOL_PNS}
_KP_NP = {pn: np.stack(_split3_np(np.kron(_resize_matrix(pn), _resize_matrix(pn))))
          for pn in _UP_PNS}

_TAPS = [(ky, kx) for ky in range(3) for kx in range(3)]


def _split3(x):
    x1 = x.astype(jnp.bfloat16).astype(jnp.float32)
    r = x - x1
    x2 = r.astype(jnp.bfloat16).astype(jnp.float32)
    x3 = (r - x2).astype(jnp.bfloat16).astype(jnp.float32)
    return x1, x2, x3


def _mm(a, b):
    return jax.lax.dot_general(a, b, (((1,), (0,)), ((), ())),
                               precision=jax.lax.Precision.DEFAULT,
                               preferred_element_type=jnp.float32)


def _wide_to_rows(x, n):
    return jnp.concatenate([x[:, i * CVAE:(i + 1) * CVAE] for i in range(IMG)],
                           axis=0)


def _rows_to_wide(x, n):
    return jnp.concatenate([x[i * n:(i + 1) * n, :] for i in range(IMG)],
                           axis=1)


def _vq_kernel(fT_ref, emb_ref, embt_ref, wr_ref, phib_ref,
               p1_ref, p2_ref, p4_ref, p8_ref,
               kp2_ref, kp4_ref, kp8_ref,
               fhat_ref, loss_ref):
    p_refs = {1: p1_ref, 2: p2_ref, 4: p4_ref, 8: p8_ref}
    kp_refs = {2: kp2_ref, 4: kp4_ref, 8: kp8_ref}
    fT = fT_ref[...]
    embt = embt_ref[...]
    emb_sq = jnp.sum(embt * embt, axis=0, keepdims=True)
    e1, e2, e3 = _split3(emb_ref[...])
    rowi = jax.lax.broadcasted_iota(jnp.int32, (NROW, 1), 0)
    ss = rowi % S
    sy = ss // W
    sx = ss % W
    f_rest = fT
    f_hat = jnp.zeros_like(fT)
    part = jnp.zeros((), jnp.float32)
    for si, pn in enumerate(V_PATCH):
        if si != SN - 1:
            Pc = p_refs[pn][...]
            x1, x2, x3 = _split3(f_rest)
            zT = (_mm(Pc, x1) + _mm(Pc, x2)) + _mm(Pc, x3)
            rows = _wide_to_rows(zT, pn * pn)
        else:
            rows = _wide_to_rows(f_rest, S)
        z2 = jnp.sum(rows * rows, axis=1, keepdims=True)
        d = (z2 + emb_sq) - 2.0 * _mm(rows, embt)
        minv = jnp.min(d, axis=1, keepdims=True)
        iota = jax.lax.broadcasted_iota(jnp.int32, d.shape, 1)
        cand = jnp.where(d == minv, iota, VOCAB)
        m = jnp.min(cand, axis=1, keepdims=True)
        onehot = (iota == m).astype(jnp.float32)
        hs = (_mm(onehot, e1) + _mm(onehot, e2)) + _mm(onehot, e3)
        if si == 0:
            hu_rows = jnp.concatenate(
                [jnp.broadcast_to(hs[i:i + 1, :], (S, CVAE)) for i in range(IMG)],
                axis=0)
        elif si != SN - 1:
            hsT = _rows_to_wide(hs, pn * pn)
            kp = kp_refs[pn]
            k1, k2, k3 = kp[0], kp[1], kp[2]
            y1, y2, y3 = _split3(hsT)
            hu = _mm(k1, y1)
            hu = hu + (_mm(k1, y2) + _mm(k2, y1))
            hu = hu + (_mm(k1, y3) + (_mm(k2, y2) + _mm(k3, y1)))
            hu_rows = _wide_to_rows(hu, S)
        else:
            hu_rows = hs
        HW = _mm(hu_rows, wr_ref[PIDX[si]])
        conv = jnp.zeros((NROW, CVAE), jnp.float32)
        for t, (ky, kx) in enumerate(_TAPS):
            rdelta = W * (ky - 1) + (kx - 1)
            oky = sy + (ky - 1)
            okx = sx + (kx - 1)
            mask = ((oky >= 0) & (oky < H) & (okx >= 0) & (okx < W))
            HWt = HW[:, t * CVAE:(t + 1) * CVAE]
            rolled = pltpu.roll(HWt, (-rdelta) % NROW, axis=0)
            conv = conv + jnp.where(mask, rolled, 0.0)
        conv = conv + phib_ref[PIDX[si]][None, :]
        hph = hu_rows * (1.0 - QRESI) + conv * QRESI
        hphT = _rows_to_wide(hph, S)
        f_hat = f_hat + hphT
        f_rest = f_rest - hphT
        diff = f_hat - fT
        part = part + jnp.sum(diff * diff)
    fhat_ref[...] = f_hat
    loss_ref[...] = jnp.full((1, 1, 128), part, jnp.float32)


def kernel(f_BChw, emb_weight, phi_w, phi_b):
    f = f_BChw.astype(jnp.float32)
    fT = f.transpose(2, 3, 0, 1).reshape(S, B * CVAE)
    emb_t = emb_weight.T
    wr = phi_w.transpose(0, 2, 3, 4, 1).reshape(NPHI, CVAE, 9 * CVAE)
    consts = ([jnp.asarray(_P_NP[pn]) for pn in _POOL_PNS]
              + [jnp.asarray(_KP_NP[pn]) for pn in _UP_PNS])
    const_specs = (
        [pl.BlockSpec(_P_NP[pn].shape, lambda i: (0, 0)) for pn in _POOL_PNS]
        + [pl.BlockSpec(_KP_NP[pn].shape, lambda i: (0, 0, 0)) for pn in _UP_PNS])
    fhatT, loss_raw = pl.pallas_call(
        _vq_kernel,
        grid=(NBLK,),
        in_specs=[
            pl.BlockSpec((S, LANES), lambda i: (0, i)),
            pl.BlockSpec((VOCAB, CVAE), lambda i: (0, 0)),
            pl.BlockSpec((CVAE, VOCAB), lambda i: (0, 0)),
            pl.BlockSpec((NPHI, CVAE, 9 * CVAE), lambda i: (0, 0, 0)),
            pl.BlockSpec((NPHI, CVAE), lambda i: (0, 0)),
        ] + const_specs,
        out_specs=[
            pl.BlockSpec((S, LANES), lambda i: (0, i)),
            pl.BlockSpec((1, 1, 128), lambda i: (i, 0, 0)),
        ],
        out_shape=[
            jax.ShapeDtypeStruct((S, B * CVAE), jnp.float32),
            jax.ShapeDtypeStruct((NBLK, 1, 128), jnp.float32),
        ],
    )(fT, emb_weight, emb_t, wr, phi_b, *consts)
    f_hat = fhatT.reshape(H, W, B, CVAE).transpose(2, 3, 0, 1)
    total = jnp.sum(loss_raw[:, 0, 0])
    loss = jnp.asarray(total * ((BETA + 1.0) / (SN * B * CVAE * S)), jnp.float32)
    f_hat = (jax.lax.stop_gradient(f_hat) - f) + f
    return f_hat, loss
